# baseline (speedup 1.0000x reference)
import functools

import jax
import jax.numpy as jnp
from jax import lax
from jax.experimental import pallas as pl
from jax.experimental.pallas import tpu as pltpu
from jax.experimental.pallas import tpu_sc as plsc

NC, NS = 2, 16
NW = NC * NS
BB = 8
NEG_ADJ = -9e15
NEG_BLK = -1e30


def _leaky(x, s=0.2):
    return jnp.where(x >= 0, x, s * x)


def _sc_gather_main(embedding, ids_h, ids_item, nbr_flat, adj_flat, num):
    N, D = embedding.shape
    P = ids_h.shape[0]
    S = num.shape[1]
    CP = P // NW
    mesh = plsc.VectorSubcoreMesh(core_axis_name="c", subcore_axis_name="s")

    @functools.partial(
        pl.kernel,
        mesh=mesh,
        out_type=[
            jax.ShapeDtypeStruct((P, D), jnp.float32),
            jax.ShapeDtypeStruct((P, D), jnp.float32),
            jax.ShapeDtypeStruct((P, S), jnp.float32),
            jax.ShapeDtypeStruct((S * P, D), jnp.float32),
        ],
        scratch_types=[
            pltpu.VMEM((CP,), jnp.int32),
            pltpu.VMEM((CP,), jnp.int32),
            pltpu.VMEM((CP, S), jnp.float32),
            pltpu.VMEM((CP, D), jnp.float32),
            pltpu.SemaphoreType.DMA,
        ],
        compiler_params=pltpu.CompilerParams(use_tc_tiling_on_sc=False),
    )
    def k(emb_hbm, idsh_hbm, idsi_hbm, nbrf_hbm, adjf_hbm, num_hbm,
          h_out, item_out, nbrw_out, ev1_out,
          idx_v, nbr_v, w_v, rows_v, sem):
        wid = lax.axis_index("s") * NC + lax.axis_index("c")
        base = wid * CP
        pltpu.sync_copy(idsh_hbm.at[pl.ds(base, CP)], idx_v)
        pltpu.async_copy(emb_hbm.at[idx_v], rows_v, sem).wait()
        pltpu.sync_copy(rows_v, h_out.at[pl.ds(base, CP)])
        pltpu.async_copy(num_hbm.at[idx_v], w_v, sem).wait()
        pltpu.sync_copy(w_v, nbrw_out.at[pl.ds(base, CP)])
        pltpu.sync_copy(idsi_hbm.at[pl.ds(base, CP)], idx_v)
        pltpu.async_copy(emb_hbm.at[idx_v], rows_v, sem).wait()
        pltpu.sync_copy(rows_v, item_out.at[pl.ds(base, CP)])
        for j in range(S):
            pltpu.sync_copy(nbrf_hbm.at[pl.ds(j * P + base, CP)], idx_v)
            pltpu.async_copy(adjf_hbm.at[idx_v], nbr_v, sem).wait()
            pltpu.async_copy(emb_hbm.at[nbr_v], rows_v, sem).wait()
            pltpu.sync_copy(rows_v, ev1_out.at[pl.ds(j * P + base, CP)])

    return k(embedding, ids_h, ids_item, nbr_flat, adj_flat, num)


def _sc_gather_perm(table, idx):
    P, D = table.shape
    CP = P // NW
    mesh = plsc.VectorSubcoreMesh(core_axis_name="c", subcore_axis_name="s")

    @functools.partial(
        pl.kernel,
        mesh=mesh,
        out_type=jax.ShapeDtypeStruct((P, D), jnp.float32),
        scratch_types=[
            pltpu.VMEM((CP,), jnp.int32),
            pltpu.VMEM((CP, D), jnp.float32),
            pltpu.SemaphoreType.DMA,
        ],
    )
    def k(tab_hbm, idx_hbm, out_hbm, idx_v, rows_v, sem):
        wid = lax.axis_index("s") * NC + lax.axis_index("c")
        base = wid * CP
        pltpu.sync_copy(idx_hbm.at[pl.ds(base, CP)], idx_v)
        pltpu.async_copy(tab_hbm.at[idx_v], rows_v, sem).wait()
        pltpu.sync_copy(rows_v, out_hbm.at[pl.ds(base, CP)])

    return k(table, idx)


def _tc_main(h, item_rows, nbrw, ev1, adjbd, maskf, A, W1a, w1b, w2, W3a, W3b):
    P, D = h.shape
    S = nbrw.shape[1]
    B, L = maskf.shape
    PB = BB * L
    NBLK = B // BB

    def body(h_ref, item_ref, nbrw_ref, ev1_ref, adj_ref, mask_ref,
             a_ref, w1a_ref, w1b_ref, w2_ref, w3a_ref, w3b_ref,
             out_ref, hl_ref, pos_ref):
        hb = h_ref[...]
        es = []
        for kk in range(4):
            ak = a_ref[kk:kk + 1, :]
            ek = lax.dot_general(hb * ak, hb, (((1,), (1,)), ((), ())),
                                 preferred_element_type=jnp.float32)
            es.append(_leaky(ek))
        c = adj_ref[0]
        alpha = jnp.full((PB, PB), NEG_ADJ, jnp.float32)
        alpha = jnp.where(c == 5, NEG_BLK, alpha)
        alpha = jnp.where(c == 1, es[0], alpha)
        alpha = jnp.where(c == 2, es[1], alpha)
        alpha = jnp.where(c == 3, es[2], alpha)
        alpha = jnp.where(c == 4, es[3], alpha)
        m = jnp.max(alpha, axis=1, keepdims=True)
        p = jnp.exp(alpha - m)
        alpha = p / jnp.sum(p, axis=1, keepdims=True)
        hl = jnp.dot(alpha, hb, preferred_element_type=jnp.float32)
        mk = mask_ref[...]
        it3 = item_ref[...].reshape(BB, L, D)
        s_item = jnp.sum(it3 * mk[:, :, None], axis=1) \
            / jnp.sum(mk, axis=1, keepdims=True)
        extra = jnp.broadcast_to(s_item[:, None, :], (BB, L, D)).reshape(PB, D)
        w1a = w1a_ref[...]
        w1b = w1b_ref[...]
        w2 = w2_ref[...]
        e_list = []
        for s in range(S):
            ev_s = ev1_ref[s]
            a = jnp.dot(extra * ev_s, w1a, preferred_element_type=jnp.float32)
            a = a + nbrw_ref[:, s:s + 1] * w1b
            a = _leaky(a)
            e_list.append(jnp.dot(a, w2, preferred_element_type=jnp.float32))
        E = jnp.concatenate(e_list, axis=1)
        mE = jnp.max(E, axis=1, keepdims=True)
        pE = jnp.exp(E - mE)
        W = pE / jnp.sum(pE, axis=1, keepdims=True)
        nvec = jnp.zeros((PB, D), jnp.float32)
        for s in range(S):
            nvec = nvec + W[:, s:s + 1] * ev1_ref[s]
        hg = jnp.dot(hb, w3a_ref[...], preferred_element_type=jnp.float32) \
            + jnp.dot(nvec, w3b_ref[...], preferred_element_type=jnp.float32)
        hg = jnp.maximum(hg, 0.0)
        out_ref[...] = hl + hg
        hl_ref[...] = hl
        pos_ref[...] = jnp.sum((hl * hg).reshape(BB, L, D), axis=1)

    full = lambda shp: pl.BlockSpec(shp, lambda i: tuple(0 for _ in shp))
    return pl.pallas_call(
        body,
        grid=(NBLK,),
        in_specs=[
            pl.BlockSpec((PB, D), lambda i: (i, 0)),
            pl.BlockSpec((PB, D), lambda i: (i, 0)),
            pl.BlockSpec((PB, S), lambda i: (i, 0)),
            pl.BlockSpec((S, PB, D), lambda i: (0, i, 0)),
            pl.BlockSpec((1, PB, PB), lambda i: (i, 0, 0)),
            pl.BlockSpec((BB, L), lambda i: (i, 0)),
            full((4, D)), full((D, D)), full((1, D)), full((D, 1)),
            full((D, D)), full((D, D)),
        ],
        out_specs=[
            pl.BlockSpec((PB, D), lambda i: (i, 0)),
            pl.BlockSpec((PB, D), lambda i: (i, 0)),
            pl.BlockSpec((BB, D), lambda i: (i, 0)),
        ],
        out_shape=[
            jax.ShapeDtypeStruct((P, D), jnp.float32),
            jax.ShapeDtypeStruct((P, D), jnp.float32),
            jax.ShapeDtypeStruct((B, D), jnp.float32),
        ],
    )(h, item_rows, nbrw, ev1, adjbd, maskf, A, W1a, w1b, w2, W3a, W3b)


def _tc_loss(out_flat, hl_flat, hlp_flat, pos):
    P, D = out_flat.shape
    B = pos.shape[0]
    L = P // B
    PB = BB * L
    NBLK = B // BB

    def body(out_ref, hl_ref, hlp_ref, pos_ref, loss_ref):
        i = pl.program_id(0)

        @pl.when(i == 0)
        def _():
            loss_ref[...] = jnp.zeros((1, 1), jnp.float32)

        hg = out_ref[...] - hl_ref[...]
        neg = jnp.sum((hg * hlp_ref[...]).reshape(BB, L, D), axis=1)
        ps = pos_ref[...]
        sig_p = 1.0 / (1.0 + jnp.exp(-ps))
        sig_n = 1.0 / (1.0 + jnp.exp(-neg))
        t = -jnp.log(1e-8 + sig_p) - jnp.log(1e-8 + 1.0 - sig_n)
        loss_ref[...] += jnp.sum(t, axis=(0, 1), keepdims=True)

    return pl.pallas_call(
        body,
        grid=(NBLK,),
        in_specs=[
            pl.BlockSpec((PB, D), lambda i: (i, 0)),
            pl.BlockSpec((PB, D), lambda i: (i, 0)),
            pl.BlockSpec((PB, D), lambda i: (i, 0)),
            pl.BlockSpec((BB, D), lambda i: (i, 0)),
        ],
        out_specs=pl.BlockSpec((1, 1), lambda i: (0, 0)),
        out_shape=jax.ShapeDtypeStruct((1, 1), jnp.float32),
    )(out_flat, hl_flat, hlp_flat, pos)


def kernel(embedding, num, a0, a1, a2, a3, g_w1, g_w2, g_w3,
           inputs, adj, mask_item, item, targets, adj_all):
    B, L = inputs.shape
    N, D = embedding.shape
    S = adj_all.shape[1]
    P = B * L
    NBLK = B // BB
    PB = BB * L

    ids_h = inputs.reshape(-1).astype(jnp.int32)
    ids_item = item.reshape(-1).astype(jnp.int32)
    nbr_flat = (jnp.arange(S, dtype=jnp.int32)[:, None]
                + ids_h[None, :] * S).reshape(-1)
    adj_flat = adj_all.reshape(-1).astype(jnp.int32)

    h, item_rows, nbrw, ev1_flat = _sc_gather_main(
        embedding, ids_h, ids_item, nbr_flat, adj_flat, num)
    ev1 = ev1_flat.reshape(S, P, D)

    adj4 = adj.astype(jnp.int32).reshape(NBLK, BB, L, 1, L)
    eye = (jnp.arange(BB)[:, None, None, None]
           == jnp.arange(BB)[None, None, :, None])
    big = jnp.where(eye, adj4, jnp.int32(5))
    adjbd = big.reshape(NBLK, PB, PB)

    maskf = mask_item.astype(jnp.float32)
    A = jnp.concatenate([a0, a1, a2, a3], axis=1).T
    W1a, w1b = g_w1[:D, :], g_w1[D:D + 1, :]
    W3a, W3b = g_w3[:D, :], g_w3[D:, :]

    out_flat, hl_flat, pos = _tc_main(
        h, item_rows, nbrw, ev1, adjbd, maskf, A, W1a, w1b, g_w2, W3a, W3b)

    kp = jax.random.key(12345)
    p_row = jax.random.permutation(jax.random.fold_in(kp, 0), B)
    p_col = jax.random.permutation(jax.random.fold_in(kp, 1), L)
    perm_flat = (p_row[:, None] * L + p_col[None, :]).reshape(-1)
    perm_flat = perm_flat.astype(jnp.int32)

    hlp_flat = _sc_gather_perm(hl_flat, perm_flat)
    loss = _tc_loss(out_flat, hl_flat, hlp_flat, pos)

    BETA = 0.005
    return out_flat.reshape(B, L, D), (BETA * loss[0, 0]).astype(jnp.float32)


# speedup vs baseline: 1.0486x; 1.0486x over previous
import functools

import jax
import jax.numpy as jnp
from jax import lax
from jax.experimental import pallas as pl
from jax.experimental.pallas import tpu as pltpu
from jax.experimental.pallas import tpu_sc as plsc

NC, NS = 2, 16
NW = NC * NS
BB = 8
NEG_ADJ = -9e15
NEG_BLK = -1e30


def _leaky(x, s=0.2):
    return jnp.where(x >= 0, x, s * x)


def _sc_gather_main(embedding, ids_h, ids_item, nbr_flat, adj_flat, num):
    N, D = embedding.shape
    P = ids_h.shape[0]
    S = num.shape[1]
    CP = P // NW
    mesh = plsc.VectorSubcoreMesh(core_axis_name="c", subcore_axis_name="s")

    @functools.partial(
        pl.kernel,
        mesh=mesh,
        out_type=[
            jax.ShapeDtypeStruct((P, D), jnp.float32),
            jax.ShapeDtypeStruct((P, D), jnp.float32),
            jax.ShapeDtypeStruct((P, S), jnp.float32),
            jax.ShapeDtypeStruct((S * P, D), jnp.float32),
        ],
        scratch_types=[
            pltpu.VMEM((CP,), jnp.int32),
            pltpu.VMEM((CP,), jnp.int32),
            pltpu.VMEM((CP, S), jnp.float32),
            pltpu.VMEM((CP, D), jnp.float32),
            pltpu.SemaphoreType.DMA,
        ],
        compiler_params=pltpu.CompilerParams(use_tc_tiling_on_sc=False),
    )
    def k(emb_hbm, idsh_hbm, idsi_hbm, nbrf_hbm, adjf_hbm, num_hbm,
          h_out, item_out, nbrw_out, ev1_out,
          idx_v, nbr_v, w_v, rows_v, sem):
        wid = lax.axis_index("s") * NC + lax.axis_index("c")
        base = wid * CP
        pltpu.sync_copy(idsh_hbm.at[pl.ds(base, CP)], idx_v)
        pltpu.async_copy(emb_hbm.at[idx_v], rows_v, sem).wait()
        pltpu.sync_copy(rows_v, h_out.at[pl.ds(base, CP)])
        pltpu.async_copy(num_hbm.at[idx_v], w_v, sem).wait()
        pltpu.sync_copy(w_v, nbrw_out.at[pl.ds(base, CP)])
        pltpu.sync_copy(idsi_hbm.at[pl.ds(base, CP)], idx_v)
        pltpu.async_copy(emb_hbm.at[idx_v], rows_v, sem).wait()
        pltpu.sync_copy(rows_v, item_out.at[pl.ds(base, CP)])
        for j in range(S):
            pltpu.sync_copy(nbrf_hbm.at[pl.ds(j * P + base, CP)], idx_v)
            pltpu.async_copy(adjf_hbm.at[idx_v], nbr_v, sem).wait()
            pltpu.async_copy(emb_hbm.at[nbr_v], rows_v, sem).wait()
            pltpu.sync_copy(rows_v, ev1_out.at[pl.ds(j * P + base, CP)])

    return k(embedding, ids_h, ids_item, nbr_flat, adj_flat, num)


def _sc_gather_perm(table, idx):
    P, D = table.shape
    CP = P // NW
    mesh = plsc.VectorSubcoreMesh(core_axis_name="c", subcore_axis_name="s")

    @functools.partial(
        pl.kernel,
        mesh=mesh,
        out_type=jax.ShapeDtypeStruct((P, D), jnp.float32),
        scratch_types=[
            pltpu.VMEM((CP,), jnp.int32),
            pltpu.VMEM((CP, D), jnp.float32),
            pltpu.SemaphoreType.DMA,
        ],
    )
    def k(tab_hbm, idx_hbm, out_hbm, idx_v, rows_v, sem):
        wid = lax.axis_index("s") * NC + lax.axis_index("c")
        base = wid * CP
        pltpu.sync_copy(idx_hbm.at[pl.ds(base, CP)], idx_v)
        pltpu.async_copy(tab_hbm.at[idx_v], rows_v, sem).wait()
        pltpu.sync_copy(rows_v, out_hbm.at[pl.ds(base, CP)])

    return k(table, idx)


def _tc_main(h, item_rows, nbrw, ev1, adj4, maskf, A, W1a, w1b, w2, W3a, W3b):
    P, D = h.shape
    S = nbrw.shape[1]
    B, L = maskf.shape
    PB = BB * L
    NBLK = B // BB

    def body(h_ref, item_ref, nbrw_ref, ev1_ref, adj_ref, mask_ref,
             a_ref, w1a_ref, w1b_ref, w2_ref, w3a_ref, w3b_ref,
             out_ref, hl_ref, pos_ref, c_ref):
        hb = h_ref[...]
        es = []
        for kk in range(4):
            ak = a_ref[kk:kk + 1, :]
            ek = lax.dot_general(hb * ak, hb, (((1,), (1,)), ((), ())),
                                 preferred_element_type=jnp.float32)
            es.append(_leaky(ek))
        c_ref[...] = jnp.full((PB, PB), 5, jnp.int32)
        for bb in range(BB):
            c_ref[bb * L:(bb + 1) * L, bb * L:(bb + 1) * L] = adj_ref[0, bb]
        c = c_ref[...]
        alpha = jnp.full((PB, PB), NEG_ADJ, jnp.float32)
        alpha = jnp.where(c == 5, NEG_BLK, alpha)
        alpha = jnp.where(c == 1, es[0], alpha)
        alpha = jnp.where(c == 2, es[1], alpha)
        alpha = jnp.where(c == 3, es[2], alpha)
        alpha = jnp.where(c == 4, es[3], alpha)
        m = jnp.max(alpha, axis=1, keepdims=True)
        p = jnp.exp(alpha - m)
        alpha = p / jnp.sum(p, axis=1, keepdims=True)
        hl = jnp.dot(alpha, hb, preferred_element_type=jnp.float32)
        mk = mask_ref[...]
        it3 = item_ref[...].reshape(BB, L, D)
        s_item = jnp.sum(it3 * mk[:, :, None], axis=1) \
            / jnp.sum(mk, axis=1, keepdims=True)
        extra = jnp.broadcast_to(s_item[:, None, :], (BB, L, D)).reshape(PB, D)
        w1a = w1a_ref[...]
        w1b = w1b_ref[...]
        w2 = w2_ref[...]
        e_list = []
        for s in range(S):
            ev_s = ev1_ref[s]
            a = jnp.dot(extra * ev_s, w1a, preferred_element_type=jnp.float32)
            a = a + nbrw_ref[:, s:s + 1] * w1b
            a = _leaky(a)
            e_list.append(jnp.dot(a, w2, preferred_element_type=jnp.float32))
        E = jnp.concatenate(e_list, axis=1)
        mE = jnp.max(E, axis=1, keepdims=True)
        pE = jnp.exp(E - mE)
        W = pE / jnp.sum(pE, axis=1, keepdims=True)
        nvec = jnp.zeros((PB, D), jnp.float32)
        for s in range(S):
            nvec = nvec + W[:, s:s + 1] * ev1_ref[s]
        hg = jnp.dot(hb, w3a_ref[...], preferred_element_type=jnp.float32) \
            + jnp.dot(nvec, w3b_ref[...], preferred_element_type=jnp.float32)
        hg = jnp.maximum(hg, 0.0)
        out_ref[...] = (hl + hg).reshape(BB, L, D)
        hl_ref[...] = hl
        pos_ref[...] = jnp.sum((hl * hg).reshape(BB, L, D), axis=1)

    full = lambda shp: pl.BlockSpec(shp, lambda i: tuple(0 for _ in shp))
    return pl.pallas_call(
        body,
        grid=(NBLK,),
        in_specs=[
            pl.BlockSpec((PB, D), lambda i: (i, 0)),
            pl.BlockSpec((PB, D), lambda i: (i, 0)),
            pl.BlockSpec((PB, S), lambda i: (i, 0)),
            pl.BlockSpec((S, PB, D), lambda i: (0, i, 0)),
            pl.BlockSpec((1, BB, L, L), lambda i: (i, 0, 0, 0)),
            pl.BlockSpec((BB, L), lambda i: (i, 0)),
            full((4, D)), full((D, D)), full((1, D)), full((D, 1)),
            full((D, D)), full((D, D)),
        ],
        out_specs=[
            pl.BlockSpec((BB, L, D), lambda i: (i, 0, 0)),
            pl.BlockSpec((PB, D), lambda i: (i, 0)),
            pl.BlockSpec((BB, D), lambda i: (i, 0)),
        ],
        out_shape=[
            jax.ShapeDtypeStruct((B, L, D), jnp.float32),
            jax.ShapeDtypeStruct((P, D), jnp.float32),
            jax.ShapeDtypeStruct((B, D), jnp.float32),
        ],
        scratch_shapes=[pltpu.VMEM((PB, PB), jnp.int32)],
    )(h, item_rows, nbrw, ev1, adj4, maskf, A, W1a, w1b, w2, W3a, W3b)


def _tc_loss(out, hl_flat, hlp_flat, pos):
    B, L, D = out.shape
    PB = BB * L
    NBLK = B // BB

    def body(out_ref, hl_ref, hlp_ref, pos_ref, loss_ref, acc_ref):
        i = pl.program_id(0)

        @pl.when(i == 0)
        def _():
            acc_ref[...] = jnp.zeros((BB, D), jnp.float32)

        hg = out_ref[...].reshape(PB, D) - hl_ref[...]
        neg = jnp.sum((hg * hlp_ref[...]).reshape(BB, L, D), axis=1)
        ps = pos_ref[...]
        sig_p = 1.0 / (1.0 + jnp.exp(-ps))
        sig_n = 1.0 / (1.0 + jnp.exp(-neg))
        t = -jnp.log(1e-8 + sig_p) - jnp.log(1e-8 + 1.0 - sig_n)
        acc_ref[...] += t

        @pl.when(i == NBLK - 1)
        def _():
            loss_ref[...] = jnp.sum(acc_ref[...], axis=(0, 1), keepdims=True)

    return pl.pallas_call(
        body,
        grid=(NBLK,),
        in_specs=[
            pl.BlockSpec((BB, L, D), lambda i: (i, 0, 0)),
            pl.BlockSpec((PB, D), lambda i: (i, 0)),
            pl.BlockSpec((PB, D), lambda i: (i, 0)),
            pl.BlockSpec((BB, D), lambda i: (i, 0)),
        ],
        out_specs=pl.BlockSpec((1, 1), lambda i: (0, 0)),
        out_shape=jax.ShapeDtypeStruct((1, 1), jnp.float32),
        scratch_shapes=[pltpu.VMEM((BB, D), jnp.float32)],
    )(out, hl_flat, hlp_flat, pos)


def kernel(embedding, num, a0, a1, a2, a3, g_w1, g_w2, g_w3,
           inputs, adj, mask_item, item, targets, adj_all):
    B, L = inputs.shape
    N, D = embedding.shape
    S = adj_all.shape[1]
    P = B * L
    NBLK = B // BB
    PB = BB * L

    ids_h = inputs.reshape(-1).astype(jnp.int32)
    ids_item = item.reshape(-1).astype(jnp.int32)
    nbr_flat = (jnp.arange(S, dtype=jnp.int32)[:, None]
                + ids_h[None, :] * S).reshape(-1)
    adj_flat = adj_all.reshape(-1).astype(jnp.int32)

    h, item_rows, nbrw, ev1_flat = _sc_gather_main(
        embedding, ids_h, ids_item, nbr_flat, adj_flat, num)
    ev1 = ev1_flat.reshape(S, P, D)

    adj4 = adj.astype(jnp.int32).reshape(NBLK, BB, L, L)
    maskf = mask_item.astype(jnp.float32)
    A = jnp.concatenate([a0, a1, a2, a3], axis=1).T
    W1a, w1b = g_w1[:D, :], g_w1[D:D + 1, :]
    W3a, W3b = g_w3[:D, :], g_w3[D:, :]

    out, hl_flat, pos = _tc_main(
        h, item_rows, nbrw, ev1, adj4, maskf, A, W1a, w1b, g_w2, W3a, W3b)

    kp = jax.random.key(12345)
    p_row = jax.random.permutation(jax.random.fold_in(kp, 0), B)
    p_col = jax.random.permutation(jax.random.fold_in(kp, 1), L)
    perm_flat = (p_row[:, None] * L + p_col[None, :]).reshape(-1)
    perm_flat = perm_flat.astype(jnp.int32)

    hlp_flat = _sc_gather_perm(hl_flat, perm_flat)
    loss = _tc_loss(out, hl_flat, hlp_flat, pos)

    BETA = 0.005
    return out, (BETA * loss[0, 0]).astype(jnp.float32)
